# baseline (speedup 1.0000x reference)
import functools

import jax
import jax.numpy as jnp
from jax import lax
from jax.experimental import pallas as pl
from jax.experimental.pallas import tpu as pltpu
from jax.experimental.pallas import tpu_sc as plsc

N = 10000
E = 320000
D = 128
H = 128

_INTERPRET = False

_NC = 2
_NS = 16
_NW = _NC * _NS

_K = 128
_G = 8
_NCH = 80
E_PAD = _NW * _K * _NCH
_NCHT = E_PAD // _K
N_PAD = 10112
_RPT = N_PAD // _NS

_mesh = plsc.VectorSubcoreMesh(core_axis_name="c", subcore_axis_name="s",
                               num_cores=_NC, num_subcores=_NS)


def _dot(a, b):
    return jnp.dot(a, b, preferred_element_type=jnp.float32)


def _deg_body(dst2_hbm, z128_hbm, ones_hbm, out0, out1, idx_d8, ones_v, dacc):
    cid = lax.axis_index("c")
    sid = lax.axis_index("s")
    wid = sid * _NC + cid
    r0 = sid * _RPT
    pltpu.sync_copy(z128_hbm.at[pl.ds(r0, _RPT)], dacc.at[pl.ds(r0, _RPT)])
    pltpu.sync_copy(ones_hbm, ones_v)
    plsc.subcore_barrier()
    cb = wid * _NCH

    def group(g, carry):
        ch = cb + g * _G
        pltpu.sync_copy(dst2_hbm.at[pl.ds(ch, _G)], idx_d8)
        for q in range(_G):
            pltpu.sync_copy(ones_v, dacc.at[idx_d8.at[q]], add=True)
        return carry

    lax.fori_loop(0, _NCH // _G, group, 0)
    plsc.subcore_barrier()

    @pl.when(cid == 0)
    def _():
        pltpu.sync_copy(dacc.at[pl.ds(r0, _RPT)], out0.at[pl.ds(r0, _RPT)])

    @pl.when(cid == 1)
    def _():
        pltpu.sync_copy(dacc.at[pl.ds(r0, _RPT)], out1.at[pl.ds(r0, _RPT)])


_deg_kernel = pl.kernel(
    _deg_body,
    out_type=(
        jax.ShapeDtypeStruct((N_PAD, H), jnp.float32),
        jax.ShapeDtypeStruct((N_PAD, H), jnp.float32),
    ),
    mesh=_mesh,
    scratch_types=[
        pltpu.VMEM((_G, _K), jnp.int32),
        pltpu.VMEM((_K, H), jnp.float32),
        pltpu.VMEM_SHARED((N_PAD, H), jnp.float32),
    ],
)


def _seg_body(hw_hbm, src2_hbm, dst2_hbm, z128_hbm, out0, out1,
              idx_s8, idx_d8, rows_a, rows_b, acc, sem_a, sem_b):
    cid = lax.axis_index("c")
    sid = lax.axis_index("s")
    wid = sid * _NC + cid
    r0 = sid * _RPT
    pltpu.sync_copy(z128_hbm.at[pl.ds(r0, _RPT)], acc.at[pl.ds(r0, _RPT)])
    plsc.subcore_barrier()
    cb = wid * _NCH
    rows = (rows_a, rows_b)
    sems = (sem_a, sem_b)

    def group(g, carry):
        ch = cb + g * _G
        pltpu.sync_copy(src2_hbm.at[pl.ds(ch, _G)], idx_s8)
        pltpu.sync_copy(dst2_hbm.at[pl.ds(ch, _G)], idx_d8)
        descs = [None, None]
        descs[0] = pltpu.async_copy(hw_hbm.at[idx_s8.at[0]], rows[0], sems[0])
        for q in range(_G):
            cur = q & 1
            if q + 1 < _G:
                descs[1 - cur] = pltpu.async_copy(
                    hw_hbm.at[idx_s8.at[q + 1]], rows[1 - cur], sems[1 - cur])
            descs[cur].wait()
            pltpu.sync_copy(rows[cur], acc.at[idx_d8.at[q]], add=True)
        return carry

    lax.fori_loop(0, _NCH // _G, group, 0)
    plsc.subcore_barrier()

    @pl.when(cid == 0)
    def _():
        pltpu.sync_copy(acc.at[pl.ds(r0, _RPT)], out0.at[pl.ds(r0, _RPT)])

    @pl.when(cid == 1)
    def _():
        pltpu.sync_copy(acc.at[pl.ds(r0, _RPT)], out1.at[pl.ds(r0, _RPT)])


_seg_kernel = pl.kernel(
    _seg_body,
    out_type=(
        jax.ShapeDtypeStruct((N_PAD, H), jnp.float32),
        jax.ShapeDtypeStruct((N_PAD, H), jnp.float32),
    ),
    mesh=_mesh,
    scratch_types=[
        pltpu.VMEM((_G, _K), jnp.int32),
        pltpu.VMEM((_G, _K), jnp.int32),
        pltpu.VMEM((_K, H), jnp.float32),
        pltpu.VMEM((_K, H), jnp.float32),
        pltpu.VMEM_SHARED((N_PAD, H), jnp.float32),
        pltpu.SemaphoreType.DMA,
        pltpu.SemaphoreType.DMA,
    ],
)


def _egather_body(t_hbm, i2_hbm, j2_hbm, ti_out, tj_out,
                  idx_i8, idx_j8, bi_a, bj_a, bi_b, bj_b, sem_a, sem_b):
    cid = lax.axis_index("c")
    sid = lax.axis_index("s")
    wid = sid * _NC + cid
    cb = wid * _NCH
    bi = (bi_a, bi_b)
    bj = (bj_a, bj_b)
    sems = (sem_a, sem_b)

    def group(g, carry):
        ch = cb + g * _G
        pltpu.sync_copy(i2_hbm.at[pl.ds(ch, _G)], idx_i8)
        pltpu.sync_copy(j2_hbm.at[pl.ds(ch, _G)], idx_j8)
        descs = [[None, None], [None, None]]
        descs[0][0] = pltpu.async_copy(t_hbm.at[idx_i8.at[0]], bi[0], sems[0])
        descs[0][1] = pltpu.async_copy(t_hbm.at[idx_j8.at[0]], bj[0], sems[0])
        for q in range(_G):
            cur = q & 1
            if q + 1 < _G:
                descs[1 - cur][0] = pltpu.async_copy(
                    t_hbm.at[idx_i8.at[q + 1]], bi[1 - cur], sems[1 - cur])
                descs[1 - cur][1] = pltpu.async_copy(
                    t_hbm.at[idx_j8.at[q + 1]], bj[1 - cur], sems[1 - cur])
            descs[cur][0].wait()
            descs[cur][1].wait()
            b = (ch + q) * _K
            pltpu.sync_copy(bi[cur], ti_out.at[pl.ds(b, _K)])
            pltpu.sync_copy(bj[cur], tj_out.at[pl.ds(b, _K)])
        return carry

    lax.fori_loop(0, _NCH // _G, group, 0)


_egather_kernel = pl.kernel(
    _egather_body,
    out_type=(
        jax.ShapeDtypeStruct((E_PAD, H), jnp.uint32),
        jax.ShapeDtypeStruct((E_PAD, H), jnp.uint32),
    ),
    mesh=_mesh,
    scratch_types=[
        pltpu.VMEM((_G, _K), jnp.int32),
        pltpu.VMEM((_G, _K), jnp.int32),
        pltpu.VMEM((_K, H), jnp.uint32),
        pltpu.VMEM((_K, H), jnp.uint32),
        pltpu.VMEM((_K, H), jnp.uint32),
        pltpu.VMEM((_K, H), jnp.uint32),
        pltpu.SemaphoreType.DMA,
        pltpu.SemaphoreType.DMA,
    ],
)


def _node_body(f_ref, wp1_ref, bp1_ref, wp2_ref, bp2_ref, winc_ref, winp_ref,
               bin_ref, ws_ref, bsn_ref, wn_ref,
               czp_ref, hs_ref, hw_ref):
    x = f_ref[...]
    n = x.shape[0]
    col = lax.broadcasted_iota(jnp.int32, (1, D), 1)
    mask = col < 3
    s1 = jnp.sum(x, axis=0, keepdims=True)
    s2 = jnp.sum(x * x, axis=0, keepdims=True)
    m = s1 / n
    var = (s2 - n * m * m) / (n - 1)
    std = jnp.sqrt(jnp.maximum(var, 0.0))
    rs = jnp.where(mask, 1.0 / (std + 1e-6), 0.0)
    mm = jnp.where(mask, m, 0.0)
    czf = (x - mm) * rs
    p1 = jax.nn.relu(_dot(x, wp1_ref[...]) + bp1_ref[...])
    p2 = jax.nn.relu(_dot(p1, wp2_ref[...]) + bp2_ref[...])
    h0 = jax.nn.relu(_dot(czf, winc_ref[...]) + _dot(p2, winp_ref[...])
                     + bin_ref[...])
    czp_ref[...] = czf[:, :16]
    hs_ref[...] = _dot(h0, ws_ref[...]) + bsn_ref[...]
    hw_ref[...] = _dot(h0, wn_ref[...])


def _node_stage(F_all, Wp1e, bp1, Wp2, bp2, Wince, Winp, b_in, Ws, bsn, Wn):
    return pl.pallas_call(
        _node_body,
        out_shape=(
            jax.ShapeDtypeStruct((N, 16), jnp.float32),
            jax.ShapeDtypeStruct((N, H), jnp.float32),
            jax.ShapeDtypeStruct((N, H), jnp.float32),
        ),
        interpret=_INTERPRET,
    )(F_all, Wp1e, bp1[None, :], Wp2, bp2[None, :], Wince, Winp,
      b_in[None, :], Ws, bsn[None, :], Wn)


def _comb_body(hs_ref, sega_ref, segb_ref, dega_ref, degb_ref,
               ws_ref, bsn_ref, wn_ref, hs2_ref, hw2_ref):
    d = dega_ref[:N] + degb_ref[:N]
    inv = 1.0 / jnp.maximum(d, 1.0)
    seg = sega_ref[:N] + segb_ref[:N]
    h = jax.nn.relu(hs_ref[...] + seg * inv)
    hs2_ref[...] = _dot(h, ws_ref[...]) + bsn_ref[...]
    hw2_ref[...] = _dot(h, wn_ref[...])


def _comb_stage(hs, sega, segb, dega, degb, Ws, bsn, Wn):
    return pl.pallas_call(
        _comb_body,
        out_shape=(
            jax.ShapeDtypeStruct((N, H), jnp.float32),
            jax.ShapeDtypeStruct((N, H), jnp.float32),
        ),
        interpret=_INTERPRET,
    )(hs, sega, segb, dega, degb, Ws, bsn[None, :], Wn)


def _comb_final_body(hs_ref, sega_ref, segb_ref, dega_ref, degb_ref,
                     czp_ref, t_ref):
    d = dega_ref[:N] + degb_ref[:N]
    inv = 1.0 / jnp.maximum(d, 1.0)
    seg = sega_ref[:N] + segb_ref[:N]
    h = jax.nn.relu(hs_ref[...] + seg * inv)
    hb = lax.bitcast_convert_type(h.astype(jnp.bfloat16), jnp.uint16)
    czx = jnp.concatenate(
        [czp_ref[...], jnp.zeros((N, H - 16), jnp.float32)], axis=1)
    cb = lax.bitcast_convert_type(czx.astype(jnp.bfloat16), jnp.uint16)
    t_ref[...] = hb.astype(jnp.uint32) | (cb.astype(jnp.uint32) << 16)


def _comb_final(hs, sega, segb, dega, degb, czp):
    return pl.pallas_call(
        _comb_final_body,
        out_shape=jax.ShapeDtypeStruct((N, H), jnp.uint32),
        interpret=_INTERPRET,
    )(hs, sega, segb, dega, degb, czp)


_EB = 2000


def _unpack(t):
    lo = lax.bitcast_convert_type(t << 16, jnp.float32)
    hi = lax.bitcast_convert_type(t & jnp.uint32(0xFFFF0000), jnp.float32)
    return lo, hi


def _edge_body(ti_ref, tj_ref, w1a_ref, w1b_ref, w1c_ref, w1d_ref,
               be1_ref, w2_ref, be2_ref, w3_ref, be3_ref, out_ref):
    bf16 = jnp.bfloat16
    hi, czi = _unpack(ti_ref[...])
    hj, czj = _unpack(tj_ref[...])
    ab = jnp.abs(hi - hj).astype(bf16)
    dz = (czi - czj).astype(bf16)
    x = (_dot(hi.astype(bf16), w1a_ref[...]) + _dot(hj.astype(bf16), w1b_ref[...])
         + _dot(ab, w1c_ref[...]) + _dot(dz, w1d_ref[...]))
    x = jax.nn.relu(x + be1_ref[...])
    x = jax.nn.relu(_dot(x.astype(bf16), w2_ref[...]) + be2_ref[...])
    out_ref[...] = jnp.sum(x * w3_ref[...], axis=1, keepdims=True) + be3_ref[...]


def _edge_stage(TI, TJ, W1a, W1b, W1c, W1de, be1, We2, be2, w3r, be3):
    grid = (E // _EB,)
    full = lambda shape: pl.BlockSpec(shape, lambda i: (0, 0))
    out = pl.pallas_call(
        _edge_body,
        grid=grid,
        in_specs=[
            pl.BlockSpec((_EB, H), lambda i: (i, 0)),
            pl.BlockSpec((_EB, H), lambda i: (i, 0)),
            full((H, 256)), full((H, 256)), full((H, 256)), full((H, 256)),
            full((1, 256)), full((256, 256)), full((1, 256)), full((1, 256)),
            full((1, 1)),
        ],
        out_specs=pl.BlockSpec((_EB, 1), lambda i: (i, 0)),
        out_shape=jax.ShapeDtypeStruct((E, 1), jnp.float32),
        interpret=_INTERPRET,
    )(TI, TJ, W1a, W1b, W1c, W1de, be1[None, :], We2, be2[None, :],
      w3r, be3.reshape(1, 1))
    return out


def kernel(F_all, edge_und, edge_dir, Wp1, bp1, Wp2, bp2, Win, b_in,
           Ws0, bs0, Wn0, bn0, Ws1, bs1, Wn1, bn1, Ws2, bs2, Wn2, bn2,
           We1, be1, We2, be2, We3, be3):
    f32 = jnp.float32
    bf16 = jnp.bfloat16
    Wp1e = jnp.pad(Wp1, ((3, 0), (0, 0)))
    Wince = jnp.pad(Win[:3], ((0, D - 3), (0, 0)))
    Winp = Win[3:]
    W1a = We1[:H].astype(bf16)
    W1b = We1[H:2 * H].astype(bf16)
    W1c = We1[2 * H:3 * H].astype(bf16)
    W1de = jnp.pad(We1[3 * H:], ((0, H - 3), (0, 0))).astype(bf16)
    We2b = We2.astype(bf16)
    w3r = We3[:, 0][None, :]

    src, dst = edge_dir[0], edge_dir[1]
    i, j = edge_und[0], edge_und[1]
    padz = jnp.zeros((E_PAD - E,), jnp.int32)
    srcp = jnp.concatenate([src, padz]).reshape(_NCHT, _K)
    dstp = jnp.concatenate(
        [dst, jnp.full((E_PAD - E,), N, jnp.int32)]).reshape(_NCHT, _K)
    ip = jnp.concatenate([i, padz]).reshape(_NCHT, _K)
    jp = jnp.concatenate([j, padz]).reshape(_NCHT, _K)
    z128 = jnp.zeros((N_PAD, H), f32)
    ones128 = jnp.ones((_K, H), f32)

    czp, hs, hw = _node_stage(F_all, Wp1e, bp1, Wp2, bp2, Wince, Winp, b_in,
                              Ws0, bs0 + bn0, Wn0)

    dega, degb = _deg_kernel(dstp, z128, ones128)
    sega, segb = _seg_kernel(hw, srcp, dstp, z128)
    hs, hw = _comb_stage(hs, sega, segb, dega, degb, Ws1, bs1 + bn1, Wn1)
    sega, segb = _seg_kernel(hw, srcp, dstp, z128)
    hs, hw = _comb_stage(hs, sega, segb, dega, degb, Ws2, bs2 + bn2, Wn2)
    sega, segb = _seg_kernel(hw, srcp, dstp, z128)
    T = _comb_final(hs, sega, segb, dega, degb, czp)

    TI, TJ = _egather_kernel(T, ip, jp)
    logits = _edge_stage(TI, TJ, W1a, W1b, W1c, W1de, be1, We2b, be2, w3r, be3)
    return logits.reshape(-1)


# speedup vs baseline: 1.0278x; 1.0278x over previous
import functools

import jax
import jax.numpy as jnp
from jax import lax
from jax.experimental import pallas as pl
from jax.experimental.pallas import tpu as pltpu
from jax.experimental.pallas import tpu_sc as plsc

N = 10000
E = 320000
D = 128
H = 128

_INTERPRET = False

_NC = 1
_NS = 16
_NW = _NC * _NS

_K = 128
_G = 8
_NCH = 160
E_PAD = _NW * _K * _NCH
_NCHT = E_PAD // _K
N_PAD = 10112
_RPT = N_PAD // _NS

_mesh = plsc.VectorSubcoreMesh(core_axis_name="c", subcore_axis_name="s",
                               num_cores=_NC, num_subcores=_NS)


def _dot(a, b):
    return jnp.dot(a, b, preferred_element_type=jnp.float32)


def _deg_body(dst2_hbm, z128_hbm, ones_hbm, out0, idx_d8, ones_v, dacc):
    sid = lax.axis_index("s")
    wid = sid
    r0 = sid * _RPT
    pltpu.sync_copy(z128_hbm.at[pl.ds(r0, _RPT)], dacc.at[pl.ds(r0, _RPT)])
    pltpu.sync_copy(ones_hbm, ones_v)
    plsc.subcore_barrier()
    cb = wid * _NCH

    def group(g, carry):
        ch = cb + g * _G
        pltpu.sync_copy(dst2_hbm.at[pl.ds(ch, _G)], idx_d8)
        for q in range(_G):
            pltpu.sync_copy(ones_v, dacc.at[idx_d8.at[q]], add=True)
        return carry

    lax.fori_loop(0, _NCH // _G, group, 0)
    plsc.subcore_barrier()
    pltpu.sync_copy(dacc.at[pl.ds(r0, _RPT)], out0.at[pl.ds(r0, _RPT)])


_deg_kernel = pl.kernel(
    _deg_body,
    out_type=jax.ShapeDtypeStruct((N_PAD, H), jnp.float32),
    mesh=_mesh,
    scratch_types=[
        pltpu.VMEM((_G, _K), jnp.int32),
        pltpu.VMEM((_K, H), jnp.float32),
        pltpu.VMEM_SHARED((N_PAD, H), jnp.float32),
    ],
)


def _seg_body(hw_hbm, src2_hbm, dst2_hbm, z128_hbm, out0,
              idx_s8, idx_d8, rows_a, rows_b, acc, sem_a, sem_b):
    sid = lax.axis_index("s")
    wid = sid
    r0 = sid * _RPT
    pltpu.sync_copy(z128_hbm.at[pl.ds(r0, _RPT)], acc.at[pl.ds(r0, _RPT)])
    plsc.subcore_barrier()
    cb = wid * _NCH
    rows = (rows_a, rows_b)
    sems = (sem_a, sem_b)

    def group(g, carry):
        ch = cb + g * _G
        pltpu.sync_copy(src2_hbm.at[pl.ds(ch, _G)], idx_s8)
        pltpu.sync_copy(dst2_hbm.at[pl.ds(ch, _G)], idx_d8)
        descs = [None, None]
        descs[0] = pltpu.async_copy(hw_hbm.at[idx_s8.at[0]], rows[0], sems[0])
        for q in range(_G):
            cur = q & 1
            if q + 1 < _G:
                descs[1 - cur] = pltpu.async_copy(
                    hw_hbm.at[idx_s8.at[q + 1]], rows[1 - cur], sems[1 - cur])
            descs[cur].wait()
            pltpu.sync_copy(rows[cur], acc.at[idx_d8.at[q]], add=True)
        return carry

    lax.fori_loop(0, _NCH // _G, group, 0)
    plsc.subcore_barrier()
    pltpu.sync_copy(acc.at[pl.ds(r0, _RPT)], out0.at[pl.ds(r0, _RPT)])


_seg_kernel = pl.kernel(
    _seg_body,
    out_type=jax.ShapeDtypeStruct((N_PAD, H), jnp.float32),
    mesh=_mesh,
    scratch_types=[
        pltpu.VMEM((_G, _K), jnp.int32),
        pltpu.VMEM((_G, _K), jnp.int32),
        pltpu.VMEM((_K, H), jnp.float32),
        pltpu.VMEM((_K, H), jnp.float32),
        pltpu.VMEM_SHARED((N_PAD, H), jnp.float32),
        pltpu.SemaphoreType.DMA,
        pltpu.SemaphoreType.DMA,
    ],
)


def _egather_body(t_hbm, i2_hbm, j2_hbm, ti_out, tj_out,
                  idx_i8, idx_j8, bi_a, bj_a, bi_b, bj_b, sem_a, sem_b):
    sid = lax.axis_index("s")
    wid = sid
    cb = wid * _NCH
    bi = (bi_a, bi_b)
    bj = (bj_a, bj_b)
    sems = (sem_a, sem_b)

    def group(g, carry):
        ch = cb + g * _G
        pltpu.sync_copy(i2_hbm.at[pl.ds(ch, _G)], idx_i8)
        pltpu.sync_copy(j2_hbm.at[pl.ds(ch, _G)], idx_j8)
        descs = [[None, None], [None, None]]
        descs[0][0] = pltpu.async_copy(t_hbm.at[idx_i8.at[0]], bi[0], sems[0])
        descs[0][1] = pltpu.async_copy(t_hbm.at[idx_j8.at[0]], bj[0], sems[0])
        for q in range(_G):
            cur = q & 1
            if q + 1 < _G:
                descs[1 - cur][0] = pltpu.async_copy(
                    t_hbm.at[idx_i8.at[q + 1]], bi[1 - cur], sems[1 - cur])
                descs[1 - cur][1] = pltpu.async_copy(
                    t_hbm.at[idx_j8.at[q + 1]], bj[1 - cur], sems[1 - cur])
            descs[cur][0].wait()
            descs[cur][1].wait()
            b = (ch + q) * _K
            pltpu.sync_copy(bi[cur], ti_out.at[pl.ds(b, _K)])
            pltpu.sync_copy(bj[cur], tj_out.at[pl.ds(b, _K)])
        return carry

    lax.fori_loop(0, _NCH // _G, group, 0)


_egather_kernel = pl.kernel(
    _egather_body,
    out_type=(
        jax.ShapeDtypeStruct((E_PAD, H), jnp.uint32),
        jax.ShapeDtypeStruct((E_PAD, H), jnp.uint32),
    ),
    mesh=_mesh,
    scratch_types=[
        pltpu.VMEM((_G, _K), jnp.int32),
        pltpu.VMEM((_G, _K), jnp.int32),
        pltpu.VMEM((_K, H), jnp.uint32),
        pltpu.VMEM((_K, H), jnp.uint32),
        pltpu.VMEM((_K, H), jnp.uint32),
        pltpu.VMEM((_K, H), jnp.uint32),
        pltpu.SemaphoreType.DMA,
        pltpu.SemaphoreType.DMA,
    ],
)


def _node_body(f_ref, wp1_ref, bp1_ref, wp2_ref, bp2_ref, winc_ref, winp_ref,
               bin_ref, ws_ref, bsn_ref, wn_ref,
               czp_ref, hs_ref, hw_ref):
    x = f_ref[...]
    n = x.shape[0]
    col = lax.broadcasted_iota(jnp.int32, (1, D), 1)
    mask = col < 3
    s1 = jnp.sum(x, axis=0, keepdims=True)
    s2 = jnp.sum(x * x, axis=0, keepdims=True)
    m = s1 / n
    var = (s2 - n * m * m) / (n - 1)
    std = jnp.sqrt(jnp.maximum(var, 0.0))
    rs = jnp.where(mask, 1.0 / (std + 1e-6), 0.0)
    mm = jnp.where(mask, m, 0.0)
    czf = (x - mm) * rs
    p1 = jax.nn.relu(_dot(x, wp1_ref[...]) + bp1_ref[...])
    p2 = jax.nn.relu(_dot(p1, wp2_ref[...]) + bp2_ref[...])
    h0 = jax.nn.relu(_dot(czf, winc_ref[...]) + _dot(p2, winp_ref[...])
                     + bin_ref[...])
    czp_ref[...] = czf[:, :16]
    hs_ref[...] = _dot(h0, ws_ref[...]) + bsn_ref[...]
    hw_ref[...] = _dot(h0, wn_ref[...])


def _node_stage(F_all, Wp1e, bp1, Wp2, bp2, Wince, Winp, b_in, Ws, bsn, Wn):
    return pl.pallas_call(
        _node_body,
        out_shape=(
            jax.ShapeDtypeStruct((N, 16), jnp.float32),
            jax.ShapeDtypeStruct((N, H), jnp.float32),
            jax.ShapeDtypeStruct((N, H), jnp.float32),
        ),
        interpret=_INTERPRET,
    )(F_all, Wp1e, bp1[None, :], Wp2, bp2[None, :], Wince, Winp,
      b_in[None, :], Ws, bsn[None, :], Wn)


def _comb_body(hs_ref, sega_ref, dega_ref,
               ws_ref, bsn_ref, wn_ref, hs2_ref, hw2_ref):
    d = dega_ref[:N]
    inv = 1.0 / jnp.maximum(d, 1.0)
    seg = sega_ref[:N]
    h = jax.nn.relu(hs_ref[...] + seg * inv)
    hs2_ref[...] = _dot(h, ws_ref[...]) + bsn_ref[...]
    hw2_ref[...] = _dot(h, wn_ref[...])


def _comb_stage(hs, sega, dega, Ws, bsn, Wn):
    return pl.pallas_call(
        _comb_body,
        out_shape=(
            jax.ShapeDtypeStruct((N, H), jnp.float32),
            jax.ShapeDtypeStruct((N, H), jnp.float32),
        ),
        interpret=_INTERPRET,
    )(hs, sega, dega, Ws, bsn[None, :], Wn)


def _comb_final_body(hs_ref, sega_ref, dega_ref, czp_ref, t_ref):
    d = dega_ref[:N]
    inv = 1.0 / jnp.maximum(d, 1.0)
    seg = sega_ref[:N]
    h = jax.nn.relu(hs_ref[...] + seg * inv)
    hb = lax.bitcast_convert_type(h.astype(jnp.bfloat16), jnp.uint16)
    czx = jnp.concatenate(
        [czp_ref[...], jnp.zeros((N, H - 16), jnp.float32)], axis=1)
    cb = lax.bitcast_convert_type(czx.astype(jnp.bfloat16), jnp.uint16)
    t_ref[...] = hb.astype(jnp.uint32) | (cb.astype(jnp.uint32) << 16)


def _comb_final(hs, sega, dega, czp):
    return pl.pallas_call(
        _comb_final_body,
        out_shape=jax.ShapeDtypeStruct((N, H), jnp.uint32),
        interpret=_INTERPRET,
    )(hs, sega, dega, czp)


_EB = 2000


def _unpack(t):
    lo = lax.bitcast_convert_type(t << 16, jnp.float32)
    hi = lax.bitcast_convert_type(t & jnp.uint32(0xFFFF0000), jnp.float32)
    return lo, hi


def _edge_body(ti_ref, tj_ref, w1a_ref, w1b_ref, w1c_ref, w1d_ref,
               be1_ref, w2_ref, be2_ref, w3_ref, be3_ref, out_ref):
    bf16 = jnp.bfloat16
    hi, czi = _unpack(ti_ref[...])
    hj, czj = _unpack(tj_ref[...])
    ab = jnp.abs(hi - hj).astype(bf16)
    dz = (czi - czj).astype(bf16)
    x = (_dot(hi.astype(bf16), w1a_ref[...]) + _dot(hj.astype(bf16), w1b_ref[...])
         + _dot(ab, w1c_ref[...]) + _dot(dz, w1d_ref[...]))
    x = jax.nn.relu(x + be1_ref[...])
    x = jax.nn.relu(_dot(x.astype(bf16), w2_ref[...]) + be2_ref[...])
    out_ref[...] = jnp.sum(x * w3_ref[...], axis=1, keepdims=True) + be3_ref[...]


def _edge_stage(TI, TJ, W1a, W1b, W1c, W1de, be1, We2, be2, w3r, be3):
    grid = (E // _EB,)
    full = lambda shape: pl.BlockSpec(shape, lambda i: (0, 0))
    out = pl.pallas_call(
        _edge_body,
        grid=grid,
        in_specs=[
            pl.BlockSpec((_EB, H), lambda i: (i, 0)),
            pl.BlockSpec((_EB, H), lambda i: (i, 0)),
            full((H, 256)), full((H, 256)), full((H, 256)), full((H, 256)),
            full((1, 256)), full((256, 256)), full((1, 256)), full((1, 256)),
            full((1, 1)),
        ],
        out_specs=pl.BlockSpec((_EB, 1), lambda i: (i, 0)),
        out_shape=jax.ShapeDtypeStruct((E, 1), jnp.float32),
        interpret=_INTERPRET,
    )(TI, TJ, W1a, W1b, W1c, W1de, be1[None, :], We2, be2[None, :],
      w3r, be3.reshape(1, 1))
    return out


def kernel(F_all, edge_und, edge_dir, Wp1, bp1, Wp2, bp2, Win, b_in,
           Ws0, bs0, Wn0, bn0, Ws1, bs1, Wn1, bn1, Ws2, bs2, Wn2, bn2,
           We1, be1, We2, be2, We3, be3):
    f32 = jnp.float32
    bf16 = jnp.bfloat16
    Wp1e = jnp.pad(Wp1, ((3, 0), (0, 0)))
    Wince = jnp.pad(Win[:3], ((0, D - 3), (0, 0)))
    Winp = Win[3:]
    W1a = We1[:H].astype(bf16)
    W1b = We1[H:2 * H].astype(bf16)
    W1c = We1[2 * H:3 * H].astype(bf16)
    W1de = jnp.pad(We1[3 * H:], ((0, H - 3), (0, 0))).astype(bf16)
    We2b = We2.astype(bf16)
    w3r = We3[:, 0][None, :]

    src, dst = edge_dir[0], edge_dir[1]
    i, j = edge_und[0], edge_und[1]
    padz = jnp.zeros((E_PAD - E,), jnp.int32)
    srcp = jnp.concatenate([src, padz]).reshape(_NCHT, _K)
    dstp = jnp.concatenate(
        [dst, jnp.full((E_PAD - E,), N, jnp.int32)]).reshape(_NCHT, _K)
    ip = jnp.concatenate([i, padz]).reshape(_NCHT, _K)
    jp = jnp.concatenate([j, padz]).reshape(_NCHT, _K)
    z128 = jnp.zeros((N_PAD, H), f32)
    ones128 = jnp.ones((_K, H), f32)

    czp, hs, hw = _node_stage(F_all, Wp1e, bp1, Wp2, bp2, Wince, Winp, b_in,
                              Ws0, bs0 + bn0, Wn0)

    dega = _deg_kernel(dstp, z128, ones128)
    sega = _seg_kernel(hw, srcp, dstp, z128)
    hs, hw = _comb_stage(hs, sega, dega, Ws1, bs1 + bn1, Wn1)
    sega = _seg_kernel(hw, srcp, dstp, z128)
    hs, hw = _comb_stage(hs, sega, dega, Ws2, bs2 + bn2, Wn2)
    sega = _seg_kernel(hw, srcp, dstp, z128)
    T = _comb_final(hs, sega, dega, czp)

    TI, TJ = _egather_kernel(T, ip, jp)
    logits = _edge_stage(TI, TJ, W1a, W1b, W1c, W1de, be1, We2b, be2, w3r, be3)
    return logits.reshape(-1)
